# Initial kernel scaffold; baseline (speedup 1.0000x reference)
import jax
import jax.numpy as jnp
from jax.experimental import pallas as pl


def kernel(pos, batch, reflectance, sf, edge_index, W1, b1, W2, b2):
    raise NotImplementedError("write your pallas kernel here")


# speedup vs baseline: 1.1214x; 1.1214x over previous
import functools

import jax
import jax.numpy as jnp
from jax import lax
from jax.experimental import pallas as pl
from jax.experimental.pallas import tpu as pltpu
from jax.experimental.pallas import tpu_sc as plsc

N = 50000
E = 800000
C_H = 45
C_OUT = 64
NEG_SLOPE = 0.01

NC, NS = 2, 16
NW = NC * NS
LANES = 16

NROW = 50016
EPAD = 802816
E_PER_W = EPAD // NW
K1_ROWS = 7
K1_CB = K1_ROWS * 128
K1_NCHUNK = E_PER_W // K1_CB
ROWS_PER_W = E_PER_W // 128
K2_BLK = 2048
K3_CB = 2048
K3_NCHUNK = EPAD // K3_CB
ACCW = 50176
NEG_INF = float("-inf")

_mesh = plsc.VectorSubcoreMesh(core_axis_name="c", subcore_axis_name="s")


def _wid():
    return lax.axis_index("s") * NC + lax.axis_index("c")


def _leaky(x):
    return jnp.where(x >= 0, x, NEG_SLOPE * x)


@functools.partial(
    pl.kernel,
    mesh=_mesh,
    out_type=(jax.ShapeDtypeStruct((EPAD, LANES), jnp.float32),
              jax.ShapeDtypeStruct((EPAD, LANES), jnp.float32)),
    scratch_types=[
        pltpu.VMEM((K1_ROWS, 128), jnp.int32),
        pltpu.VMEM((K1_ROWS, 128), jnp.int32),
        pltpu.VMEM((K1_CB, LANES), jnp.float32),
        pltpu.VMEM((K1_CB, LANES), jnp.float32),
        pltpu.SemaphoreType.DMA,
    ],
    compiler_params=pltpu.CompilerParams(use_tc_tiling_on_sc=False,
                                        needs_layout_passes=False),
)
def _k1_gather(table_hbm, src_hbm, dst_hbm, srow_hbm, drow_hbm,
               isrc_v, idst_v, sv, dv, sem):
    w = _wid()

    def chunk(k, _):
        rbase = w * ROWS_PER_W + k * K1_ROWS
        pltpu.sync_copy(src_hbm.at[pl.ds(rbase, K1_ROWS)], isrc_v)
        pltpu.sync_copy(dst_hbm.at[pl.ds(rbase, K1_ROWS)], idst_v)
        descs = []
        for j in range(K1_ROWS):
            descs.append(pltpu.async_copy(
                table_hbm.at[isrc_v.at[j]],
                sv.at[pl.ds(j * 128, 128)], sem))
            descs.append(pltpu.async_copy(
                table_hbm.at[idst_v.at[j]],
                dv.at[pl.ds(j * 128, 128)], sem))
        for d in descs:
            d.wait()
        off = w * E_PER_W + k * K1_CB
        pltpu.sync_copy(sv, srow_hbm.at[pl.ds(off, K1_CB)])
        pltpu.sync_copy(dv, drow_hbm.at[pl.ds(off, K1_CB)])
        return 0

    lax.fori_loop(0, K1_NCHUNK, chunk, 0)


def _k2_body(s_ref, d_ref, w1p_ref, b1_ref, w2_ref, sf_ref, h_ref):
    recip = 1.0 / sf_ref[0]
    row = lax.broadcasted_iota(jnp.int32, (LANES, 1), 0)
    w1 = w1p_ref[...] * jnp.where(row < 3, recip, 1.0)
    rel = s_ref[...] - d_ref[...]
    a1 = jnp.dot(rel, w1, preferred_element_type=jnp.float32)
    g = _leaky(a1 + b1_ref[...])
    a2 = jnp.dot(g, w2_ref[...], preferred_element_type=jnp.float32)
    h_ref[...] = a2.T[:, None, :]


_k2_mlp = pl.pallas_call(
    _k2_body,
    grid=(EPAD // K2_BLK,),
    in_specs=[
        pl.BlockSpec((K2_BLK, LANES), lambda i: (i, 0)),
        pl.BlockSpec((K2_BLK, LANES), lambda i: (i, 0)),
        pl.BlockSpec((LANES, C_H), lambda i: (0, 0)),
        pl.BlockSpec((1, C_H), lambda i: (0, 0)),
        pl.BlockSpec((C_H, C_OUT), lambda i: (0, 0)),
        pl.BlockSpec(memory_space=pltpu.SMEM),
    ],
    out_specs=pl.BlockSpec((C_OUT, 1, K2_BLK), lambda i: (0, 0, i)),
    out_shape=jax.ShapeDtypeStruct((C_OUT, 1, EPAD), jnp.float32),
)


@functools.partial(
    pl.kernel,
    mesh=_mesh,
    out_type=jax.ShapeDtypeStruct((C_OUT, 1, ACCW), jnp.float32),
    scratch_types=[
        pltpu.VMEM((2 * ACCW,), jnp.float32),
        pltpu.VMEM((K3_CB,), jnp.int32),
        pltpu.VMEM((2, K3_CB), jnp.float32),
        pltpu.SemaphoreType.DMA,
    ],
    compiler_params=pltpu.CompilerParams(use_tc_tiling_on_sc=False,
                                        needs_layout_passes=False),
)
def _k3_segmax(dst_hbm, h_hbm, acc_hbm, acc_v, idx_v, h_v, sem):
    c0 = _wid() * 2
    ninf = jnp.full((LANES,), NEG_INF, jnp.float32)
    full = jnp.full((LANES,), True)

    def init(i, _):
        acc_v[pl.ds(i * LANES, LANES)] = ninf
        return 0

    lax.fori_loop(0, 2 * ACCW // LANES, init, 0)

    def chunk(k, _):
        off = k * K3_CB
        pltpu.sync_copy(dst_hbm.at[pl.ds(off, K3_CB)], idx_v)
        pltpu.sync_copy(h_hbm.at[c0, 0, pl.ds(off, K3_CB)], h_v.at[0])
        pltpu.sync_copy(h_hbm.at[c0 + 1, 0, pl.ds(off, K3_CB)], h_v.at[1])

        def group(g, _):
            idx = idx_v[pl.ds(g * LANES, LANES)]
            for c in range(2):
                fidx = idx + (c * ACCW)
                hv = h_v[c, pl.ds(g * LANES, LANES)]

                def attempt(m):
                    cur = plsc.load_gather(acc_v, [fidx])
                    plsc.store_scatter(acc_v, [fidx],
                                       jnp.maximum(cur, hv), mask=m)
                    chk = plsc.load_gather(acc_v, [fidx])
                    return m & (chk < hv)

                m = attempt(full)
                lax.while_loop(lambda m: jnp.any(m), attempt, m)
            return 0

        lax.fori_loop(0, K3_CB // LANES, group, 0)
        return 0

    lax.fori_loop(0, K3_NCHUNK, chunk, 0)
    pltpu.sync_copy(acc_v.at[pl.ds(0, ACCW)], acc_hbm.at[c0, 0])
    pltpu.sync_copy(acc_v.at[pl.ds(ACCW, ACCW)], acc_hbm.at[c0 + 1, 0])


def _k4_body(acc_ref, b2_ref, out_ref):
    a = acc_ref[:, 0, :]
    y = _leaky(a + b2_ref[...])
    y = jnp.where(a == NEG_INF, 0.0, y)
    out_ref[...] = y.T


_K4_BLK = 2048
_k4_finish = pl.pallas_call(
    _k4_body,
    grid=(pl.cdiv(N, _K4_BLK),),
    in_specs=[
        pl.BlockSpec((C_OUT, 1, _K4_BLK), lambda i: (0, 0, i)),
        pl.BlockSpec((C_OUT, 1), lambda i: (0, 0)),
    ],
    out_specs=pl.BlockSpec((_K4_BLK, C_OUT), lambda i: (i, 0)),
    out_shape=jax.ShapeDtypeStruct((N, C_OUT), jnp.float32),
)


def kernel(pos, batch, reflectance, sf, edge_index, W1, b1, W2, b2):
    table = jnp.concatenate(
        [pos, reflectance[:, None], jnp.zeros((N, LANES - 4), jnp.float32)],
        axis=1)
    table = jnp.concatenate(
        [table, jnp.zeros((NROW - N, LANES), jnp.float32)], axis=0)
    pad = jnp.full((EPAD - E,), N, jnp.int32)
    src = jnp.concatenate([edge_index[0], pad])
    dst = jnp.concatenate([edge_index[1], pad])
    src2 = src.reshape(EPAD // 128, 128)
    dst2 = dst.reshape(EPAD // 128, 128)

    W1pad = jnp.concatenate([W1, jnp.zeros((LANES - 4, C_H), jnp.float32)])

    srow, drow = _k1_gather(table, src2, dst2)
    h = _k2_mlp(srow, drow, W1pad, b1[None, :], W2, sf)
    acc = _k3_segmax(dst, h)
    return _k4_finish(acc, b2[:, None])


# speedup vs baseline: 1.7585x; 1.5682x over previous
import functools

import jax
import jax.numpy as jnp
from jax import lax
from jax.experimental import pallas as pl
from jax.experimental.pallas import tpu as pltpu
from jax.experimental.pallas import tpu_sc as plsc

N = 50000
E = 800000
C_H = 45
C_OUT = 64
NEG_SLOPE = 0.01

NC, NS = 2, 16
NW = NC * NS
LANES = 16

NROW = 50016
EPAD = 802816
E_PER_W = EPAD // NW
K1_ROWS = 7
K1_CB = K1_ROWS * 128
K1_NCHUNK = E_PER_W // K1_CB
ROWS_PER_W = E_PER_W // 128
K2_BLK = 2048
K3_CB = 2048
K3_NCHUNK = EPAD // K3_CB
ACCW = 50176
NEG_INF = float("-inf")

_mesh = plsc.VectorSubcoreMesh(core_axis_name="c", subcore_axis_name="s")


def _wid():
    return lax.axis_index("s") * NC + lax.axis_index("c")


def _leaky(x):
    return jnp.where(x >= 0, x, NEG_SLOPE * x)


@functools.partial(
    pl.kernel,
    mesh=_mesh,
    out_type=(jax.ShapeDtypeStruct((EPAD, LANES), jnp.float32),
              jax.ShapeDtypeStruct((EPAD, LANES), jnp.float32)),
    scratch_types=[
        pltpu.VMEM((K1_ROWS, 128), jnp.int32),
        pltpu.VMEM((K1_ROWS, 128), jnp.int32),
        pltpu.VMEM((K1_CB, LANES), jnp.float32),
        pltpu.VMEM((K1_CB, LANES), jnp.float32),
        pltpu.SemaphoreType.DMA,
    ],
    compiler_params=pltpu.CompilerParams(use_tc_tiling_on_sc=False,
                                        needs_layout_passes=False),
)
def _k1_gather(table_hbm, src_hbm, dst_hbm, srow_hbm, drow_hbm,
               isrc_v, idst_v, sv, dv, sem):
    w = _wid()

    def chunk(k, _):
        rbase = w * ROWS_PER_W + k * K1_ROWS
        pltpu.sync_copy(src_hbm.at[pl.ds(rbase, K1_ROWS)], isrc_v)
        pltpu.sync_copy(dst_hbm.at[pl.ds(rbase, K1_ROWS)], idst_v)
        descs = []
        for j in range(K1_ROWS):
            descs.append(pltpu.async_copy(
                table_hbm.at[isrc_v.at[j]],
                sv.at[pl.ds(j * 128, 128)], sem))
            descs.append(pltpu.async_copy(
                table_hbm.at[idst_v.at[j]],
                dv.at[pl.ds(j * 128, 128)], sem))
        for d in descs:
            d.wait()
        off = w * E_PER_W + k * K1_CB
        pltpu.sync_copy(sv, srow_hbm.at[pl.ds(off, K1_CB)])
        pltpu.sync_copy(dv, drow_hbm.at[pl.ds(off, K1_CB)])
        return 0

    lax.fori_loop(0, K1_NCHUNK, chunk, 0)


def _k2_body(s_ref, d_ref, w1p_ref, b1_ref, w2_ref, sf_ref, h_ref):
    recip = 1.0 / sf_ref[0]
    row = lax.broadcasted_iota(jnp.int32, (LANES, 1), 0)
    w1 = w1p_ref[...] * jnp.where(row < 3, recip, 1.0)
    rel = s_ref[...] - d_ref[...]
    a1 = jnp.dot(rel, w1, preferred_element_type=jnp.float32)
    g = _leaky(a1 + b1_ref[...])
    a2 = jnp.dot(g, w2_ref[...], preferred_element_type=jnp.float32)
    h_ref[...] = a2.T[:, None, :]


_k2_mlp = pl.pallas_call(
    _k2_body,
    grid=(EPAD // K2_BLK,),
    in_specs=[
        pl.BlockSpec((K2_BLK, LANES), lambda i: (i, 0)),
        pl.BlockSpec((K2_BLK, LANES), lambda i: (i, 0)),
        pl.BlockSpec((LANES, C_H), lambda i: (0, 0)),
        pl.BlockSpec((1, C_H), lambda i: (0, 0)),
        pl.BlockSpec((C_H, C_OUT), lambda i: (0, 0)),
        pl.BlockSpec(memory_space=pltpu.SMEM),
    ],
    out_specs=pl.BlockSpec((C_OUT, 1, K2_BLK), lambda i: (0, 0, i)),
    out_shape=jax.ShapeDtypeStruct((C_OUT, 1, EPAD), jnp.float32),
)


@functools.partial(
    pl.kernel,
    mesh=_mesh,
    out_type=jax.ShapeDtypeStruct((C_OUT, 1, ACCW), jnp.float32),
    scratch_types=[
        pltpu.VMEM((ACCW,), jnp.float32),
        pltpu.VMEM((ACCW,), jnp.float32),
        pltpu.VMEM((K3_CB,), jnp.int32),
        pltpu.VMEM((2, K3_CB), jnp.float32),
        pltpu.SemaphoreType.DMA,
    ],
    compiler_params=pltpu.CompilerParams(use_tc_tiling_on_sc=False,
                                        needs_layout_passes=False),
)
def _k3_segmax(dst_hbm, h_hbm, acc_hbm, acc0_v, acc1_v, idx_v, h_v, sem):
    c0 = _wid() * 2
    accs = (acc0_v, acc1_v)
    ninf = jnp.full((LANES,), NEG_INF, jnp.float32)
    full = jnp.full((LANES,), True)
    nofail = jnp.full((LANES,), False)

    def init(i, _):
        acc0_v[pl.ds(i * LANES, LANES)] = ninf
        acc1_v[pl.ds(i * LANES, LANES)] = ninf
        return 0

    lax.fori_loop(0, ACCW // LANES, init, 0)

    def chunk(k, _):
        off = k * K3_CB
        pltpu.sync_copy(dst_hbm.at[pl.ds(off, K3_CB)], idx_v)
        pltpu.sync_copy(h_hbm.at[c0, 0, pl.ds(off, K3_CB)], h_v.at[0])
        pltpu.sync_copy(h_hbm.at[c0 + 1, 0, pl.ds(off, K3_CB)], h_v.at[1])

        def group_a(g, _):
            idx = idx_v[pl.ds(g * LANES, LANES)]
            for c in range(2):
                hv = h_v[c, pl.ds(g * LANES, LANES)]
                cur = plsc.load_gather(accs[c], [idx])
                plsc.store_scatter(accs[c], [idx], jnp.maximum(cur, hv))
            return 0

        lax.fori_loop(0, K3_CB // LANES, group_a, 0)

        def group_b(g, fail):
            idx = idx_v[pl.ds(g * LANES, LANES)]
            for c in range(2):
                hv = h_v[c, pl.ds(g * LANES, LANES)]
                chk = plsc.load_gather(accs[c], [idx])
                fail = fail | (chk < hv)
            return fail

        fail = lax.fori_loop(0, K3_CB // LANES, group_b, nofail)

        @pl.when(jnp.any(fail))
        def _():
            def group_c(g, _):
                idx = idx_v[pl.ds(g * LANES, LANES)]
                hv0 = h_v[0, pl.ds(g * LANES, LANES)]
                hv1 = h_v[1, pl.ds(g * LANES, LANES)]

                def attempt(ms):
                    m0, m1 = ms
                    out = []
                    for c, (m, hv) in enumerate(((m0, hv0), (m1, hv1))):
                        cur = plsc.load_gather(accs[c], [idx])
                        plsc.store_scatter(accs[c], [idx],
                                           jnp.maximum(cur, hv), mask=m)
                        chk = plsc.load_gather(accs[c], [idx])
                        out.append(m & (chk < hv))
                    return tuple(out)

                ms = attempt((full, full))
                lax.while_loop(lambda ms: jnp.any(ms[0] | ms[1]),
                               attempt, ms)
                return 0

            lax.fori_loop(0, K3_CB // LANES, group_c, 0)

        return 0

    lax.fori_loop(0, K3_NCHUNK, chunk, 0)
    pltpu.sync_copy(acc0_v, acc_hbm.at[c0, 0])
    pltpu.sync_copy(acc1_v, acc_hbm.at[c0 + 1, 0])


def _k4_body(acc_ref, b2_ref, out_ref):
    a = acc_ref[:, 0, :]
    y = _leaky(a + b2_ref[...])
    y = jnp.where(a == NEG_INF, 0.0, y)
    out_ref[...] = y.T


_K4_BLK = 2048
_k4_finish = pl.pallas_call(
    _k4_body,
    grid=(pl.cdiv(N, _K4_BLK),),
    in_specs=[
        pl.BlockSpec((C_OUT, 1, _K4_BLK), lambda i: (0, 0, i)),
        pl.BlockSpec((C_OUT, 1), lambda i: (0, 0)),
    ],
    out_specs=pl.BlockSpec((_K4_BLK, C_OUT), lambda i: (i, 0)),
    out_shape=jax.ShapeDtypeStruct((N, C_OUT), jnp.float32),
)


def kernel(pos, batch, reflectance, sf, edge_index, W1, b1, W2, b2):
    table = jnp.concatenate(
        [pos, reflectance[:, None], jnp.zeros((N, LANES - 4), jnp.float32)],
        axis=1)
    table = jnp.concatenate(
        [table, jnp.zeros((NROW - N, LANES), jnp.float32)], axis=0)
    pad = jnp.full((EPAD - E,), N, jnp.int32)
    src = jnp.concatenate([edge_index[0], pad])
    dst = jnp.concatenate([edge_index[1], pad])
    src2 = src.reshape(EPAD // 128, 128)
    dst2 = dst.reshape(EPAD // 128, 128)

    W1pad = jnp.concatenate([W1, jnp.zeros((LANES - 4, C_H), jnp.float32)])

    srow, drow = _k1_gather(table, src2, dst2)
    h = _k2_mlp(srow, drow, W1pad, b1[None, :], W2, sf)
    acc = _k3_segmax(dst, h)
    return _k4_finish(acc, b2[:, None])


# speedup vs baseline: 1.8859x; 1.0725x over previous
import functools

import jax
import jax.numpy as jnp
from jax import lax
from jax.experimental import pallas as pl
from jax.experimental.pallas import tpu as pltpu
from jax.experimental.pallas import tpu_sc as plsc

N = 50000
E = 800000
C_H = 45
C_OUT = 64
NEG_SLOPE = 0.01

NC, NS = 2, 16
NW = NC * NS
LANES = 16

NROW = 50016
EPAD = 802816
E_PER_W = EPAD // NW
K1_ROWS = 7
K1_CB = K1_ROWS * 128
K1_NCHUNK = E_PER_W // K1_CB
ROWS_PER_W = E_PER_W // 128
K2_BLK = 2048
K3_CB = 2048
K3_NCHUNK = EPAD // K3_CB
ACCW = 50176
NEG_INF = float("-inf")

_mesh = plsc.VectorSubcoreMesh(core_axis_name="c", subcore_axis_name="s")


def _wid():
    return lax.axis_index("s") * NC + lax.axis_index("c")


def _leaky(x):
    return jnp.where(x >= 0, x, NEG_SLOPE * x)


@functools.partial(
    pl.kernel,
    mesh=_mesh,
    out_type=(jax.ShapeDtypeStruct((EPAD, LANES), jnp.float32),
              jax.ShapeDtypeStruct((EPAD, LANES), jnp.float32)),
    scratch_types=[
        pltpu.VMEM((K1_ROWS, 128), jnp.int32),
        pltpu.VMEM((K1_ROWS, 128), jnp.int32),
        pltpu.VMEM((K1_CB, LANES), jnp.float32),
        pltpu.VMEM((K1_CB, LANES), jnp.float32),
        pltpu.SemaphoreType.DMA,
    ],
    compiler_params=pltpu.CompilerParams(use_tc_tiling_on_sc=False,
                                        needs_layout_passes=False),
)
def _k1_gather(table_hbm, src_hbm, dst_hbm, srow_hbm, drow_hbm,
               isrc_v, idst_v, sv, dv, sem):
    w = _wid()

    def chunk(k, _):
        rbase = w * ROWS_PER_W + k * K1_ROWS
        pltpu.sync_copy(src_hbm.at[pl.ds(rbase, K1_ROWS)], isrc_v)
        pltpu.sync_copy(dst_hbm.at[pl.ds(rbase, K1_ROWS)], idst_v)
        descs = []
        for j in range(K1_ROWS):
            descs.append(pltpu.async_copy(
                table_hbm.at[isrc_v.at[j]],
                sv.at[pl.ds(j * 128, 128)], sem))
            descs.append(pltpu.async_copy(
                table_hbm.at[idst_v.at[j]],
                dv.at[pl.ds(j * 128, 128)], sem))
        for d in descs:
            d.wait()
        off = w * E_PER_W + k * K1_CB
        pltpu.sync_copy(sv, srow_hbm.at[pl.ds(off, K1_CB)])
        pltpu.sync_copy(dv, drow_hbm.at[pl.ds(off, K1_CB)])
        return 0

    lax.fori_loop(0, K1_NCHUNK, chunk, 0)


def _k2_body(s_ref, d_ref, w1p_ref, b1_ref, w2_ref, sf_ref, h_ref):
    recip = 1.0 / sf_ref[0]
    row = lax.broadcasted_iota(jnp.int32, (LANES, 1), 0)
    w1 = w1p_ref[...] * jnp.where(row < 3, recip, 1.0)
    rel = s_ref[...] - d_ref[...]
    a1 = jnp.dot(rel, w1, preferred_element_type=jnp.float32)
    g = _leaky(a1 + b1_ref[...])
    a2 = jnp.dot(g, w2_ref[...], preferred_element_type=jnp.float32)
    h_ref[...] = a2.T[:, None, :]


_k2_mlp = pl.pallas_call(
    _k2_body,
    grid=(EPAD // K2_BLK,),
    in_specs=[
        pl.BlockSpec((K2_BLK, LANES), lambda i: (i, 0)),
        pl.BlockSpec((K2_BLK, LANES), lambda i: (i, 0)),
        pl.BlockSpec((LANES, C_H), lambda i: (0, 0)),
        pl.BlockSpec((1, C_H), lambda i: (0, 0)),
        pl.BlockSpec((C_H, C_OUT), lambda i: (0, 0)),
        pl.BlockSpec(memory_space=pltpu.SMEM),
    ],
    out_specs=pl.BlockSpec((C_OUT, 1, K2_BLK), lambda i: (0, 0, i)),
    out_shape=jax.ShapeDtypeStruct((C_OUT, 1, EPAD), jnp.float32),
)


_KF_CPB = 8
_KF_G = _KF_CPB * (K3_CB // LANES)


def _kf_body(dst_ref, flag_ref):
    idx = dst_ref[...]
    cnt = jnp.zeros(idx.shape, jnp.int32)
    for j in range(LANES):
        cnt = cnt + (idx == idx[:, j:j + 1]).astype(jnp.int32)
    grp = jnp.any(cnt >= 2, axis=1)
    chunk = jnp.any(grp.reshape(_KF_CPB, K3_CB // LANES), axis=1)
    flag_ref[...] = jnp.broadcast_to(
        chunk[:, None], (_KF_CPB, LANES)).astype(jnp.int32)


_kf_flags = pl.pallas_call(
    _kf_body,
    grid=(K3_NCHUNK // _KF_CPB,),
    in_specs=[pl.BlockSpec((_KF_G, LANES), lambda i: (i, 0))],
    out_specs=pl.BlockSpec((_KF_CPB, LANES), lambda i: (i, 0)),
    out_shape=jax.ShapeDtypeStruct((K3_NCHUNK, LANES), jnp.int32),
)


@functools.partial(
    pl.kernel,
    mesh=_mesh,
    out_type=jax.ShapeDtypeStruct((C_OUT, 1, ACCW), jnp.float32),
    scratch_types=[
        pltpu.VMEM((ACCW,), jnp.float32),
        pltpu.VMEM((ACCW,), jnp.float32),
        pltpu.VMEM((K3_CB,), jnp.int32),
        pltpu.VMEM((2, K3_CB), jnp.float32),
        pltpu.VMEM((K3_NCHUNK * LANES,), jnp.int32),
        pltpu.SemaphoreType.DMA,
    ],
    compiler_params=pltpu.CompilerParams(use_tc_tiling_on_sc=False,
                                        needs_layout_passes=False),
)
def _k3_segmax(dst_hbm, h_hbm, flags_hbm, acc_hbm,
               acc0_v, acc1_v, idx_v, h_v, flags_v, sem):
    c0 = _wid() * 2
    accs = (acc0_v, acc1_v)
    ninf = jnp.full((LANES,), NEG_INF, jnp.float32)
    full = jnp.full((LANES,), True)
    nofail = jnp.full((LANES,), False)

    pltpu.sync_copy(flags_hbm, flags_v)

    def init(i, _):
        acc0_v[pl.ds(i * LANES, LANES)] = ninf
        acc1_v[pl.ds(i * LANES, LANES)] = ninf
        return 0

    lax.fori_loop(0, ACCW // LANES, init, 0)

    def chunk(k, _):
        off = k * K3_CB
        pltpu.sync_copy(dst_hbm.at[pl.ds(off, K3_CB)], idx_v)
        pltpu.sync_copy(h_hbm.at[c0, 0, pl.ds(off, K3_CB)], h_v.at[0])
        pltpu.sync_copy(h_hbm.at[c0 + 1, 0, pl.ds(off, K3_CB)], h_v.at[1])
        dirty = jnp.any(flags_v[pl.ds(k * LANES, LANES)] != 0)

        def group_a(g, _):
            idx = idx_v[pl.ds(g * LANES, LANES)]
            for c in range(2):
                hv = h_v[c, pl.ds(g * LANES, LANES)]
                cur = plsc.load_gather(accs[c], [idx])
                plsc.store_scatter(accs[c], [idx], jnp.maximum(cur, hv))
            return 0

        lax.fori_loop(0, K3_CB // LANES, group_a, 0)

        @pl.when(dirty)
        def _():
            def group_b(g, fail):
                idx = idx_v[pl.ds(g * LANES, LANES)]
                for c in range(2):
                    hv = h_v[c, pl.ds(g * LANES, LANES)]
                    chk = plsc.load_gather(accs[c], [idx])
                    fail = fail | (chk < hv)
                return fail

            fail = lax.fori_loop(0, K3_CB // LANES, group_b, nofail)

            @pl.when(jnp.any(fail))
            def _():
                def group_c(g, _):
                    idx = idx_v[pl.ds(g * LANES, LANES)]
                    hv0 = h_v[0, pl.ds(g * LANES, LANES)]
                    hv1 = h_v[1, pl.ds(g * LANES, LANES)]

                    def attempt(ms):
                        m0, m1 = ms
                        out = []
                        for c, (m, hv) in enumerate(((m0, hv0), (m1, hv1))):
                            cur = plsc.load_gather(accs[c], [idx])
                            plsc.store_scatter(accs[c], [idx],
                                               jnp.maximum(cur, hv), mask=m)
                            chk = plsc.load_gather(accs[c], [idx])
                            out.append(m & (chk < hv))
                        return tuple(out)

                    ms = attempt((full, full))
                    lax.while_loop(lambda ms: jnp.any(ms[0] | ms[1]),
                                   attempt, ms)
                    return 0

                lax.fori_loop(0, K3_CB // LANES, group_c, 0)

        return 0

    lax.fori_loop(0, K3_NCHUNK, chunk, 0)
    pltpu.sync_copy(acc0_v, acc_hbm.at[c0, 0])
    pltpu.sync_copy(acc1_v, acc_hbm.at[c0 + 1, 0])


def _k4_body(acc_ref, b2_ref, out_ref):
    a = acc_ref[:, 0, :]
    y = _leaky(a + b2_ref[...])
    y = jnp.where(a == NEG_INF, 0.0, y)
    out_ref[...] = y.T


_K4_BLK = 2048
_k4_finish = pl.pallas_call(
    _k4_body,
    grid=(pl.cdiv(N, _K4_BLK),),
    in_specs=[
        pl.BlockSpec((C_OUT, 1, _K4_BLK), lambda i: (0, 0, i)),
        pl.BlockSpec((C_OUT, 1), lambda i: (0, 0)),
    ],
    out_specs=pl.BlockSpec((_K4_BLK, C_OUT), lambda i: (i, 0)),
    out_shape=jax.ShapeDtypeStruct((N, C_OUT), jnp.float32),
)


def kernel(pos, batch, reflectance, sf, edge_index, W1, b1, W2, b2):
    table = jnp.concatenate(
        [pos, reflectance[:, None], jnp.zeros((N, LANES - 4), jnp.float32)],
        axis=1)
    table = jnp.concatenate(
        [table, jnp.zeros((NROW - N, LANES), jnp.float32)], axis=0)
    pad = jnp.full((EPAD - E,), N, jnp.int32)
    src = jnp.concatenate([edge_index[0], pad])
    dst = jnp.concatenate([edge_index[1], pad])
    src2 = src.reshape(EPAD // 128, 128)
    dst2 = dst.reshape(EPAD // 128, 128)

    W1pad = jnp.concatenate([W1, jnp.zeros((LANES - 4, C_H), jnp.float32)])

    srow, drow = _k1_gather(table, src2, dst2)
    flags = _kf_flags(dst.reshape(EPAD // LANES, LANES)).reshape(-1)
    h = _k2_mlp(srow, drow, W1pad, b1[None, :], W2, sf)
    acc = _k3_segmax(dst, h, flags)
    return _k4_finish(acc, b2[:, None])


# speedup vs baseline: 2.0131x; 1.0674x over previous
import functools

import jax
import jax.numpy as jnp
from jax import lax
from jax.experimental import pallas as pl
from jax.experimental.pallas import tpu as pltpu
from jax.experimental.pallas import tpu_sc as plsc

N = 50000
E = 800000
C_H = 45
C_OUT = 64
NEG_SLOPE = 0.01

NC, NS = 2, 16
NW = NC * NS
LANES = 16

NROW = 50016
EPAD = 802816
E_PER_W = EPAD // NW
K1_ROWS = 7
K1_CB = K1_ROWS * 128
K1_NCHUNK = E_PER_W // K1_CB
ROWS_PER_W = E_PER_W // 128
K2_BLK = 8192
K3_CB = 2048
K3_NCHUNK = EPAD // K3_CB
ACCW = 50176
NEG_INF = float("-inf")

_mesh = plsc.VectorSubcoreMesh(core_axis_name="c", subcore_axis_name="s")


def _wid():
    return lax.axis_index("s") * NC + lax.axis_index("c")


def _leaky(x):
    return jnp.where(x >= 0, x, NEG_SLOPE * x)


@functools.partial(
    pl.kernel,
    mesh=_mesh,
    out_type=(jax.ShapeDtypeStruct((EPAD, LANES), jnp.float32),
              jax.ShapeDtypeStruct((EPAD, LANES), jnp.float32)),
    scratch_types=[
        pltpu.VMEM((K1_ROWS, 128), jnp.int32),
        pltpu.VMEM((K1_ROWS, 128), jnp.int32),
        pltpu.VMEM((K1_CB, LANES), jnp.float32),
        pltpu.VMEM((K1_CB, LANES), jnp.float32),
        pltpu.SemaphoreType.DMA,
    ],
    compiler_params=pltpu.CompilerParams(use_tc_tiling_on_sc=False,
                                        needs_layout_passes=False),
)
def _k1_gather(table_hbm, src_hbm, dst_hbm, srow_hbm, drow_hbm,
               isrc_v, idst_v, sv, dv, sem):
    w = _wid()

    def chunk(k, _):
        rbase = w * ROWS_PER_W + k * K1_ROWS
        pltpu.sync_copy(src_hbm.at[pl.ds(rbase, K1_ROWS)], isrc_v)
        pltpu.sync_copy(dst_hbm.at[pl.ds(rbase, K1_ROWS)], idst_v)
        descs = []
        for j in range(K1_ROWS):
            descs.append(pltpu.async_copy(
                table_hbm.at[isrc_v.at[j]],
                sv.at[pl.ds(j * 128, 128)], sem))
            descs.append(pltpu.async_copy(
                table_hbm.at[idst_v.at[j]],
                dv.at[pl.ds(j * 128, 128)], sem))
        for d in descs:
            d.wait()
        off = w * E_PER_W + k * K1_CB
        pltpu.sync_copy(sv, srow_hbm.at[pl.ds(off, K1_CB)])
        pltpu.sync_copy(dv, drow_hbm.at[pl.ds(off, K1_CB)])
        return 0

    lax.fori_loop(0, K1_NCHUNK, chunk, 0)


def _k2_body(s_ref, d_ref, w1p_ref, b1_ref, w2_ref, sf_ref, h_ref):
    recip = 1.0 / sf_ref[0]
    row = lax.broadcasted_iota(jnp.int32, (LANES, 1), 0)
    w1 = w1p_ref[...] * jnp.where(row < 3, recip, 1.0)
    rel = s_ref[...] - d_ref[...]
    a1 = jnp.dot(rel, w1, preferred_element_type=jnp.float32)
    g = _leaky(a1 + b1_ref[...])
    a2 = jnp.dot(g, w2_ref[...], preferred_element_type=jnp.float32)
    h_ref[...] = a2.T[:, None, :]


_k2_mlp = pl.pallas_call(
    _k2_body,
    grid=(EPAD // K2_BLK,),
    in_specs=[
        pl.BlockSpec((K2_BLK, LANES), lambda i: (i, 0)),
        pl.BlockSpec((K2_BLK, LANES), lambda i: (i, 0)),
        pl.BlockSpec((LANES, C_H), lambda i: (0, 0)),
        pl.BlockSpec((1, C_H), lambda i: (0, 0)),
        pl.BlockSpec((C_H, C_OUT), lambda i: (0, 0)),
        pl.BlockSpec(memory_space=pltpu.SMEM),
    ],
    out_specs=pl.BlockSpec((C_OUT, 1, K2_BLK), lambda i: (0, 0, i)),
    out_shape=jax.ShapeDtypeStruct((C_OUT, 1, EPAD), jnp.float32),
)


_KF_CPB = 8
_KF_G = _KF_CPB * (K3_CB // LANES)


def _kf_body(dst_ref, flag_ref):
    idx = dst_ref[...]
    cnt = jnp.zeros(idx.shape, jnp.int32)
    for j in range(LANES):
        cnt = cnt + (idx == idx[:, j:j + 1]).astype(jnp.int32)
    grp = jnp.any(cnt >= 2, axis=1)
    chunk = jnp.any(grp.reshape(_KF_CPB, K3_CB // LANES), axis=1)
    flag_ref[...] = jnp.broadcast_to(
        chunk[:, None], (_KF_CPB, LANES)).astype(jnp.int32)


_kf_flags = pl.pallas_call(
    _kf_body,
    grid=(K3_NCHUNK // _KF_CPB,),
    in_specs=[pl.BlockSpec((_KF_G, LANES), lambda i: (i, 0))],
    out_specs=pl.BlockSpec((_KF_CPB, LANES), lambda i: (i, 0)),
    out_shape=jax.ShapeDtypeStruct((K3_NCHUNK, LANES), jnp.int32),
)


@functools.partial(
    pl.kernel,
    mesh=_mesh,
    out_type=jax.ShapeDtypeStruct((C_OUT, 1, ACCW), jnp.float32),
    scratch_types=[
        pltpu.VMEM((ACCW,), jnp.float32),
        pltpu.VMEM((ACCW,), jnp.float32),
        pltpu.VMEM((K3_CB,), jnp.int32),
        pltpu.VMEM((2, K3_CB), jnp.float32),
        pltpu.VMEM((K3_NCHUNK * LANES,), jnp.int32),
        pltpu.SemaphoreType.DMA,
    ],
    compiler_params=pltpu.CompilerParams(use_tc_tiling_on_sc=False,
                                        needs_layout_passes=False),
)
def _k3_segmax(dst_hbm, h_hbm, flags_hbm, acc_hbm,
               acc0_v, acc1_v, idx_v, h_v, flags_v, sem):
    c0 = _wid() * 2
    accs = (acc0_v, acc1_v)
    ninf = jnp.full((LANES,), NEG_INF, jnp.float32)
    full = jnp.full((LANES,), True)
    nofail = jnp.full((LANES,), False)

    pltpu.sync_copy(flags_hbm, flags_v)

    def init(i, _):
        acc0_v[pl.ds(i * LANES, LANES)] = ninf
        acc1_v[pl.ds(i * LANES, LANES)] = ninf
        return 0

    lax.fori_loop(0, ACCW // LANES, init, 0)

    def chunk(k, _):
        off = k * K3_CB
        pltpu.sync_copy(dst_hbm.at[pl.ds(off, K3_CB)], idx_v)
        pltpu.sync_copy(h_hbm.at[c0, 0, pl.ds(off, K3_CB)], h_v.at[0])
        pltpu.sync_copy(h_hbm.at[c0 + 1, 0, pl.ds(off, K3_CB)], h_v.at[1])
        dirty = jnp.any(flags_v[pl.ds(k * LANES, LANES)] != 0)

        def group_a(g, _):
            idx = idx_v[pl.ds(g * LANES, LANES)]
            for c in range(2):
                hv = h_v[c, pl.ds(g * LANES, LANES)]
                cur = plsc.load_gather(accs[c], [idx])
                plsc.store_scatter(accs[c], [idx], jnp.maximum(cur, hv))
            return 0

        lax.fori_loop(0, K3_CB // LANES, group_a, 0)

        @pl.when(dirty)
        def _():
            def group_b(g, fail):
                idx = idx_v[pl.ds(g * LANES, LANES)]
                for c in range(2):
                    hv = h_v[c, pl.ds(g * LANES, LANES)]
                    chk = plsc.load_gather(accs[c], [idx])
                    fail = fail | (chk < hv)
                return fail

            fail = lax.fori_loop(0, K3_CB // LANES, group_b, nofail)

            @pl.when(jnp.any(fail))
            def _():
                def group_c(g, _):
                    idx = idx_v[pl.ds(g * LANES, LANES)]
                    hv0 = h_v[0, pl.ds(g * LANES, LANES)]
                    hv1 = h_v[1, pl.ds(g * LANES, LANES)]

                    def attempt(ms):
                        m0, m1 = ms
                        out = []
                        for c, (m, hv) in enumerate(((m0, hv0), (m1, hv1))):
                            cur = plsc.load_gather(accs[c], [idx])
                            plsc.store_scatter(accs[c], [idx],
                                               jnp.maximum(cur, hv), mask=m)
                            chk = plsc.load_gather(accs[c], [idx])
                            out.append(m & (chk < hv))
                        return tuple(out)

                    ms = attempt((full, full))
                    lax.while_loop(lambda ms: jnp.any(ms[0] | ms[1]),
                                   attempt, ms)
                    return 0

                lax.fori_loop(0, K3_CB // LANES, group_c, 0)

        return 0

    lax.fori_loop(0, K3_NCHUNK, chunk, 0)
    pltpu.sync_copy(acc0_v, acc_hbm.at[c0, 0])
    pltpu.sync_copy(acc1_v, acc_hbm.at[c0 + 1, 0])


def _k4_body(acc_ref, b2_ref, out_ref):
    a = acc_ref[:, 0, :]
    y = _leaky(a + b2_ref[...])
    y = jnp.where(a == NEG_INF, 0.0, y)
    out_ref[...] = y.T


_K4_BLK = 2048
_k4_finish = pl.pallas_call(
    _k4_body,
    grid=(pl.cdiv(N, _K4_BLK),),
    in_specs=[
        pl.BlockSpec((C_OUT, 1, _K4_BLK), lambda i: (0, 0, i)),
        pl.BlockSpec((C_OUT, 1), lambda i: (0, 0)),
    ],
    out_specs=pl.BlockSpec((_K4_BLK, C_OUT), lambda i: (i, 0)),
    out_shape=jax.ShapeDtypeStruct((N, C_OUT), jnp.float32),
)


def kernel(pos, batch, reflectance, sf, edge_index, W1, b1, W2, b2):
    table = jnp.concatenate(
        [pos, reflectance[:, None], jnp.zeros((N, LANES - 4), jnp.float32)],
        axis=1)
    table = jnp.concatenate(
        [table, jnp.zeros((NROW - N, LANES), jnp.float32)], axis=0)
    pad = jnp.full((EPAD - E,), N, jnp.int32)
    src = jnp.concatenate([edge_index[0], pad])
    dst = jnp.concatenate([edge_index[1], pad])
    src2 = src.reshape(EPAD // 128, 128)
    dst2 = dst.reshape(EPAD // 128, 128)

    W1pad = jnp.concatenate([W1, jnp.zeros((LANES - 4, C_H), jnp.float32)])

    srow, drow = _k1_gather(table, src2, dst2)
    flags = _kf_flags(dst.reshape(EPAD // LANES, LANES)).reshape(-1)
    h = _k2_mlp(srow, drow, W1pad, b1[None, :], W2, sf)
    acc = _k3_segmax(dst, h, flags)
    return _k4_finish(acc, b2[:, None])


# speedup vs baseline: 2.0161x; 1.0015x over previous
import functools

import jax
import jax.numpy as jnp
from jax import lax
from jax.experimental import pallas as pl
from jax.experimental.pallas import tpu as pltpu
from jax.experimental.pallas import tpu_sc as plsc

N = 50000
E = 800000
C_H = 45
C_OUT = 64
NEG_SLOPE = 0.01

NC, NS = 2, 16
NW = NC * NS
LANES = 16

NROW = 50016
EPAD = 802816
E_PER_W = EPAD // NW
K1_ROWS = 7
K1_CB = K1_ROWS * 128
K1_NCHUNK = E_PER_W // K1_CB
ROWS_PER_W = E_PER_W // 128
K2_BLK = 16384
K3_CB = 2048
K3_NCHUNK = EPAD // K3_CB
ACCW = 50176
NEG_INF = float("-inf")

_mesh = plsc.VectorSubcoreMesh(core_axis_name="c", subcore_axis_name="s")


def _wid():
    return lax.axis_index("s") * NC + lax.axis_index("c")


def _leaky(x):
    return jnp.where(x >= 0, x, NEG_SLOPE * x)


@functools.partial(
    pl.kernel,
    mesh=_mesh,
    out_type=(jax.ShapeDtypeStruct((EPAD, LANES), jnp.float32),
              jax.ShapeDtypeStruct((EPAD, LANES), jnp.float32)),
    scratch_types=[
        pltpu.VMEM((K1_ROWS, 128), jnp.int32),
        pltpu.VMEM((K1_ROWS, 128), jnp.int32),
        pltpu.VMEM((K1_CB, LANES), jnp.float32),
        pltpu.VMEM((K1_CB, LANES), jnp.float32),
        pltpu.SemaphoreType.DMA,
    ],
    compiler_params=pltpu.CompilerParams(use_tc_tiling_on_sc=False,
                                        needs_layout_passes=False),
)
def _k1_gather(table_hbm, src_hbm, dst_hbm, srow_hbm, drow_hbm,
               isrc_v, idst_v, sv, dv, sem):
    w = _wid()

    def chunk(k, _):
        rbase = w * ROWS_PER_W + k * K1_ROWS
        pltpu.sync_copy(src_hbm.at[pl.ds(rbase, K1_ROWS)], isrc_v)
        pltpu.sync_copy(dst_hbm.at[pl.ds(rbase, K1_ROWS)], idst_v)
        descs = []
        for j in range(K1_ROWS):
            descs.append(pltpu.async_copy(
                table_hbm.at[isrc_v.at[j]],
                sv.at[pl.ds(j * 128, 128)], sem))
            descs.append(pltpu.async_copy(
                table_hbm.at[idst_v.at[j]],
                dv.at[pl.ds(j * 128, 128)], sem))
        for d in descs:
            d.wait()
        off = w * E_PER_W + k * K1_CB
        pltpu.sync_copy(sv, srow_hbm.at[pl.ds(off, K1_CB)])
        pltpu.sync_copy(dv, drow_hbm.at[pl.ds(off, K1_CB)])
        return 0

    lax.fori_loop(0, K1_NCHUNK, chunk, 0)


def _k2_body(s_ref, d_ref, w1p_ref, b1_ref, w2_ref, sf_ref, h_ref):
    recip = 1.0 / sf_ref[0]
    row = lax.broadcasted_iota(jnp.int32, (LANES, 1), 0)
    w1 = w1p_ref[...] * jnp.where(row < 3, recip, 1.0)
    rel = s_ref[...] - d_ref[...]
    a1 = jnp.dot(rel, w1, preferred_element_type=jnp.float32)
    g = _leaky(a1 + b1_ref[...])
    a2 = jnp.dot(g, w2_ref[...], preferred_element_type=jnp.float32)
    h_ref[...] = a2.T[:, None, :]


_k2_mlp = pl.pallas_call(
    _k2_body,
    grid=(EPAD // K2_BLK,),
    in_specs=[
        pl.BlockSpec((K2_BLK, LANES), lambda i: (i, 0)),
        pl.BlockSpec((K2_BLK, LANES), lambda i: (i, 0)),
        pl.BlockSpec((LANES, C_H), lambda i: (0, 0)),
        pl.BlockSpec((1, C_H), lambda i: (0, 0)),
        pl.BlockSpec((C_H, C_OUT), lambda i: (0, 0)),
        pl.BlockSpec(memory_space=pltpu.SMEM),
    ],
    out_specs=pl.BlockSpec((C_OUT, 1, K2_BLK), lambda i: (0, 0, i)),
    out_shape=jax.ShapeDtypeStruct((C_OUT, 1, EPAD), jnp.float32),
)


_KF_CPB = 8
_KF_G = _KF_CPB * (K3_CB // LANES)


def _kf_body(dst_ref, flag_ref):
    idx = dst_ref[...]
    cnt = jnp.zeros(idx.shape, jnp.int32)
    for j in range(LANES):
        cnt = cnt + (idx == idx[:, j:j + 1]).astype(jnp.int32)
    grp = jnp.any(cnt >= 2, axis=1)
    chunk = jnp.any(grp.reshape(_KF_CPB, K3_CB // LANES), axis=1)
    flag_ref[...] = jnp.broadcast_to(
        chunk[:, None], (_KF_CPB, LANES)).astype(jnp.int32)


_kf_flags = pl.pallas_call(
    _kf_body,
    grid=(K3_NCHUNK // _KF_CPB,),
    in_specs=[pl.BlockSpec((_KF_G, LANES), lambda i: (i, 0))],
    out_specs=pl.BlockSpec((_KF_CPB, LANES), lambda i: (i, 0)),
    out_shape=jax.ShapeDtypeStruct((K3_NCHUNK, LANES), jnp.int32),
)


@functools.partial(
    pl.kernel,
    mesh=_mesh,
    out_type=jax.ShapeDtypeStruct((C_OUT, 1, ACCW), jnp.float32),
    scratch_types=[
        pltpu.VMEM((ACCW,), jnp.float32),
        pltpu.VMEM((ACCW,), jnp.float32),
        pltpu.VMEM((K3_CB,), jnp.int32),
        pltpu.VMEM((2, K3_CB), jnp.float32),
        pltpu.VMEM((K3_NCHUNK * LANES,), jnp.int32),
        pltpu.SemaphoreType.DMA,
    ],
    compiler_params=pltpu.CompilerParams(use_tc_tiling_on_sc=False,
                                        needs_layout_passes=False),
)
def _k3_segmax(dst_hbm, h_hbm, flags_hbm, acc_hbm,
               acc0_v, acc1_v, idx_v, h_v, flags_v, sem):
    c0 = _wid() * 2
    accs = (acc0_v, acc1_v)
    ninf = jnp.full((LANES,), NEG_INF, jnp.float32)
    full = jnp.full((LANES,), True)
    nofail = jnp.full((LANES,), False)

    pltpu.sync_copy(flags_hbm, flags_v)

    def init(i, _):
        acc0_v[pl.ds(i * LANES, LANES)] = ninf
        acc1_v[pl.ds(i * LANES, LANES)] = ninf
        return 0

    lax.fori_loop(0, ACCW // LANES, init, 0)

    def chunk(k, _):
        off = k * K3_CB
        pltpu.sync_copy(dst_hbm.at[pl.ds(off, K3_CB)], idx_v)
        pltpu.sync_copy(h_hbm.at[c0, 0, pl.ds(off, K3_CB)], h_v.at[0])
        pltpu.sync_copy(h_hbm.at[c0 + 1, 0, pl.ds(off, K3_CB)], h_v.at[1])
        dirty = jnp.any(flags_v[pl.ds(k * LANES, LANES)] != 0)

        def group_a(g, _):
            idx = idx_v[pl.ds(g * LANES, LANES)]
            for c in range(2):
                hv = h_v[c, pl.ds(g * LANES, LANES)]
                cur = plsc.load_gather(accs[c], [idx])
                plsc.store_scatter(accs[c], [idx], jnp.maximum(cur, hv))
            return 0

        lax.fori_loop(0, K3_CB // LANES, group_a, 0)

        @pl.when(dirty)
        def _():
            def group_b(g, fail):
                idx = idx_v[pl.ds(g * LANES, LANES)]
                for c in range(2):
                    hv = h_v[c, pl.ds(g * LANES, LANES)]
                    chk = plsc.load_gather(accs[c], [idx])
                    fail = fail | (chk < hv)
                return fail

            fail = lax.fori_loop(0, K3_CB // LANES, group_b, nofail)

            @pl.when(jnp.any(fail))
            def _():
                def group_c(g, _):
                    idx = idx_v[pl.ds(g * LANES, LANES)]
                    hv0 = h_v[0, pl.ds(g * LANES, LANES)]
                    hv1 = h_v[1, pl.ds(g * LANES, LANES)]

                    def attempt(ms):
                        m0, m1 = ms
                        out = []
                        for c, (m, hv) in enumerate(((m0, hv0), (m1, hv1))):
                            cur = plsc.load_gather(accs[c], [idx])
                            plsc.store_scatter(accs[c], [idx],
                                               jnp.maximum(cur, hv), mask=m)
                            chk = plsc.load_gather(accs[c], [idx])
                            out.append(m & (chk < hv))
                        return tuple(out)

                    ms = attempt((full, full))
                    lax.while_loop(lambda ms: jnp.any(ms[0] | ms[1]),
                                   attempt, ms)
                    return 0

                lax.fori_loop(0, K3_CB // LANES, group_c, 0)

        return 0

    lax.fori_loop(0, K3_NCHUNK, chunk, 0)
    pltpu.sync_copy(acc0_v, acc_hbm.at[c0, 0])
    pltpu.sync_copy(acc1_v, acc_hbm.at[c0 + 1, 0])


def _k4_body(acc_ref, b2_ref, out_ref):
    a = acc_ref[:, 0, :]
    y = _leaky(a + b2_ref[...])
    y = jnp.where(a == NEG_INF, 0.0, y)
    out_ref[...] = y.T


_K4_BLK = 2048
_k4_finish = pl.pallas_call(
    _k4_body,
    grid=(pl.cdiv(N, _K4_BLK),),
    in_specs=[
        pl.BlockSpec((C_OUT, 1, _K4_BLK), lambda i: (0, 0, i)),
        pl.BlockSpec((C_OUT, 1), lambda i: (0, 0)),
    ],
    out_specs=pl.BlockSpec((_K4_BLK, C_OUT), lambda i: (i, 0)),
    out_shape=jax.ShapeDtypeStruct((N, C_OUT), jnp.float32),
)


def kernel(pos, batch, reflectance, sf, edge_index, W1, b1, W2, b2):
    table = jnp.concatenate(
        [pos, reflectance[:, None], jnp.zeros((N, LANES - 4), jnp.float32)],
        axis=1)
    table = jnp.concatenate(
        [table, jnp.zeros((NROW - N, LANES), jnp.float32)], axis=0)
    pad = jnp.full((EPAD - E,), N, jnp.int32)
    src = jnp.concatenate([edge_index[0], pad])
    dst = jnp.concatenate([edge_index[1], pad])
    src2 = src.reshape(EPAD // 128, 128)
    dst2 = dst.reshape(EPAD // 128, 128)

    W1pad = jnp.concatenate([W1, jnp.zeros((LANES - 4, C_H), jnp.float32)])

    srow, drow = _k1_gather(table, src2, dst2)
    flags = _kf_flags(dst.reshape(EPAD // LANES, LANES)).reshape(-1)
    h = _k2_mlp(srow, drow, W1pad, b1[None, :], W2, sf)
    acc = _k3_segmax(dst, h, flags)
    return _k4_finish(acc, b2[:, None])
